# Initial kernel scaffold; baseline (speedup 1.0000x reference)
import jax
import jax.numpy as jnp
from jax.experimental import pallas as pl


def kernel(x, edge_index, graph_ids, W1, b1, W2, b2, eps, W_head, b_head):
    raise NotImplementedError("write your pallas kernel here")


# speedup vs baseline: 2.4587x; 2.4587x over previous
import functools

import jax
import jax.numpy as jnp
from jax import lax
from jax.experimental import pallas as pl
from jax.experimental.pallas import tpu as pltpu
from jax.experimental.pallas import tpu_sc as plsc

N = 10000
E = 160000
D = 300
H = 600
L = 5
G = 64
OUT = 2048

PD = 128
NPART = 3
NP_ = 10240
NC = 2
NS = 16
CH = 128
EPT = 10240
NCHUNK = EPT // CH
SEG = 40
ZROWS = NP_ // NS
B_TC = 512
C_HD = 1280


def _gather_scatter_pass(table, sidx, didx, acc, buf_a, buf_b, sem_a, sem_b,
                         lo, n_chunks):
    pltpu.async_copy(table.at[sidx.at[lo]], buf_a, sem_a)

    def pair(j2, _):
        ch0 = lo + 2 * j2
        pltpu.make_async_copy(table.at[sidx.at[ch0]], buf_a, sem_a).wait()
        pltpu.async_copy(table.at[sidx.at[ch0 + 1]], buf_b, sem_b)
        pltpu.sync_copy(buf_a, acc.at[didx.at[ch0]], add=True)

        pltpu.make_async_copy(table.at[sidx.at[ch0 + 1]], buf_b, sem_b).wait()

        @pl.when(2 * j2 + 2 < n_chunks)
        def _():
            pltpu.async_copy(table.at[sidx.at[ch0 + 2]], buf_a, sem_a)

        pltpu.sync_copy(buf_b, acc.at[didx.at[ch0 + 1]], add=True)
        return 0

    lax.fori_loop(0, n_chunks // 2, pair, 0)


def _sc_body(h_hbm, src_hbm, dst_hbm, zero_hbm, agg_hbm, agg2_hbm,
             sidx, didx, buf_a, buf_b, acc, sem_a, sem_b):
    c = lax.axis_index("c")
    s = lax.axis_index("s")
    rows = pl.ds(s * ZROWS, ZROWS)

    def full_pass(part, lo, n_chunks, out_view):
        pltpu.sync_copy(zero_hbm, acc.at[rows])
        plsc.subcore_barrier()
        for seg in range(n_chunks // SEG):
            base = lo + seg * SEG
            pltpu.sync_copy(src_hbm.at[s, pl.ds(base, SEG)], sidx)
            pltpu.sync_copy(dst_hbm.at[s, pl.ds(base, SEG)], didx)
            _gather_scatter_pass(h_hbm.at[part], sidx, didx, acc,
                                 buf_a, buf_b, sem_a, sem_b, 0, SEG)
        plsc.subcore_barrier()
        pltpu.sync_copy(acc.at[rows], out_view.at[rows])

    @pl.when(c == 0)
    def _():
        full_pass(0, 0, NCHUNK, agg_hbm.at[0])
        full_pass(2, 0, NCHUNK // 2, agg2_hbm.at[0])

    @pl.when(c == 1)
    def _():
        full_pass(1, 0, NCHUNK, agg_hbm.at[1])
        full_pass(2, NCHUNK // 2, NCHUNK // 2, agg2_hbm.at[1])


@functools.cache
def _sc_msg_kernel():
    return pl.kernel(
        _sc_body,
        out_type=(
            jax.ShapeDtypeStruct((2, NP_, PD), jnp.float32),
            jax.ShapeDtypeStruct((2, NP_, PD), jnp.float32),
        ),
        mesh=plsc.VectorSubcoreMesh(
            core_axis_name="c", subcore_axis_name="s",
            num_cores=NC, num_subcores=NS),
        scratch_types=[
            pltpu.VMEM((SEG, CH), jnp.int32),
            pltpu.VMEM((SEG, CH), jnp.int32),
            pltpu.VMEM((CH, PD), jnp.float32),
            pltpu.VMEM((CH, PD), jnp.float32),
            pltpu.VMEM_SHARED((NP_, PD), jnp.float32),
            pltpu.SemaphoreType.DMA,
            pltpu.SemaphoreType.DMA,
        ],
    )


def _sc_msg(h, src_t, dst_t, zrows):
    return _sc_msg_kernel()(h, src_t, dst_t, zrows)


def _mlp_body(scale_ref, h_ref, agg_ref, agg2_ref, w1_ref, b1_ref, w2_ref,
              b2_ref, out_ref):
    scale = scale_ref[0, 0]
    hin = [scale * h_ref[0] + agg_ref[0],
           scale * h_ref[1] + agg_ref[1],
           scale * h_ref[2] + agg2_ref[0] + agg2_ref[1]]
    t = b1_ref[...]
    for p in range(NPART):
        t = t + jnp.dot(hin[p], w1_ref[p], preferred_element_type=jnp.float32)
    t = jnp.maximum(t, 0.0)
    for p in range(NPART):
        out_ref[p] = jnp.maximum(
            jnp.dot(t, w2_ref[p], preferred_element_type=jnp.float32)
            + b2_ref[p][None, :], 0.0)


def _mlp_call(scale, h, agg, agg2, w1, b1, w2, b2):
    return pl.pallas_call(
        _mlp_body,
        grid=(NP_ // B_TC,),
        in_specs=[
            pl.BlockSpec(memory_space=pltpu.SMEM),
            pl.BlockSpec((NPART, B_TC, PD), lambda i: (0, i, 0)),
            pl.BlockSpec((2, B_TC, PD), lambda i: (0, i, 0)),
            pl.BlockSpec((2, B_TC, PD), lambda i: (0, i, 0)),
            pl.BlockSpec((NPART, PD, H), lambda i: (0, 0, 0)),
            pl.BlockSpec((1, H), lambda i: (0, 0)),
            pl.BlockSpec((NPART, H, PD), lambda i: (0, 0, 0)),
            pl.BlockSpec((NPART, PD), lambda i: (0, 0)),
        ],
        out_specs=pl.BlockSpec((NPART, B_TC, PD), lambda i: (0, i, 0)),
        out_shape=jax.ShapeDtypeStruct((NPART, NP_, PD), jnp.float32),
    )(scale, h, agg, agg2, w1, b1, w2, b2)


def _head_body(h_ref, p_ref, wh_ref, bh_ref, out_ref, accs):
    i = pl.program_id(0)

    @pl.when(i == 0)
    def _():
        accs[...] = jnp.zeros_like(accs)

    for p in range(NPART):
        accs[p] += jnp.dot(p_ref[...], h_ref[p],
                           preferred_element_type=jnp.float32)

    @pl.when(i == NP_ // C_HD - 1)
    def _():
        cnt = accs[NPART - 1][:, PD - 1:PD]
        inv = 1.0 / jnp.maximum(cnt, 1.0)
        out = bh_ref[...]
        for p in range(NPART):
            out = out + jnp.dot(accs[p] * inv, wh_ref[p],
                                preferred_element_type=jnp.float32)
        out_ref[...] = out


def _head_call(h, p, wh, bh):
    return pl.pallas_call(
        _head_body,
        grid=(NP_ // C_HD,),
        in_specs=[
            pl.BlockSpec((NPART, C_HD, PD), lambda i: (0, i, 0)),
            pl.BlockSpec((G, C_HD), lambda i: (0, i)),
            pl.BlockSpec((NPART, PD, OUT), lambda i: (0, 0, 0)),
            pl.BlockSpec((1, OUT), lambda i: (0, 0)),
        ],
        out_specs=pl.BlockSpec((G, OUT), lambda i: (0, 0)),
        out_shape=jax.ShapeDtypeStruct((G, OUT), jnp.float32),
        scratch_shapes=[
            pltpu.VMEM((NPART, G, PD), jnp.float32),
        ],
    )(h, p, wh, bh)


def _part_pad(a, ncols_axis=-1):
    pads = [(0, 0)] * a.ndim
    pads[ncols_axis] = (0, NPART * PD - D)
    ap = jnp.pad(a, pads)
    return ap


def kernel(x, edge_index, graph_ids, W1, b1, W2, b2, eps, W_head, b_head):
    f32 = jnp.float32
    src = edge_index[0].astype(jnp.int32)
    dst = edge_index[1].astype(jnp.int32)

    src_t = jnp.full((NS, EPT), NP_ - 1, jnp.int32).at[:, :E // NS].set(
        src.reshape(NS, E // NS)).reshape(NS, NCHUNK, CH)
    dst_t = jnp.full((NS, EPT), N, jnp.int32).at[:, :E // NS].set(
        dst.reshape(NS, E // NS)).reshape(NS, NCHUNK, CH)

    xp = _part_pad(x)
    h = jnp.zeros((NPART, NP_, PD), f32)
    for p in range(NPART):
        h = h.at[p, :N].set(xp[:, p * PD:(p + 1) * PD])

    W1p = _part_pad(W1, 1).reshape(L, NPART, PD, H)
    W2p = _part_pad(W2).reshape(L, H, NPART, PD).transpose(0, 2, 1, 3)
    b2p = _part_pad(b2).reshape(L, NPART, PD)
    b1r = b1.reshape(L, 1, H)
    scales = (1.0 + eps).astype(f32).reshape(L, 1, 1)
    zrows = jnp.zeros((ZROWS, PD), f32)

    for l in range(L):
        agg, agg2 = _sc_msg(h, src_t, dst_t, zrows)
        h = _mlp_call(scales[l], h, agg, agg2,
                      W1p[l], b1r[l], W2p[l], b2p[l])

    onehot = (graph_ids[None, :] == jnp.arange(G, dtype=graph_ids.dtype)[:, None])
    pmat = jnp.zeros((G, NP_), f32).at[:, :N].set(onehot.astype(f32))
    hh = h.at[NPART - 1, :, PD - 1].set(1.0)
    whp = _part_pad(W_head, 0).reshape(NPART, PD, OUT)
    return _head_call(hh, pmat, whp, b_head.reshape(1, OUT))


# speedup vs baseline: 2.4740x; 1.0062x over previous
import functools

import jax
import jax.numpy as jnp
from jax import lax
from jax.experimental import pallas as pl
from jax.experimental.pallas import tpu as pltpu
from jax.experimental.pallas import tpu_sc as plsc

N = 10000
E = 160000
D = 300
H = 600
L = 5
G = 64
OUT = 2048

PD = 128
NPART = 3
NP_ = 10240
NC = 2
NS = 16
CH = 64
EPT = 10240
NCHUNK = EPT // CH
SEG = 40
ZROWS = NP_ // NS
B_TC = 512
C_HD = 1280


def _gather_scatter_segment(table, sidx, didx, acc, bufs, gsems, ssems):
    n = SEG
    pltpu.async_copy(table.at[sidx.at[0]], bufs[0], gsems[0])
    pltpu.async_copy(table.at[sidx.at[1]], bufs[1], gsems[1])

    def quad(i, _):
        for u in range(4):
            k = 4 * i + u
            v = (u + 2) % 4
            pltpu.make_async_copy(table.at[sidx.at[k]], bufs[u], gsems[u]).wait()
            pltpu.async_copy(bufs[u], acc.at[didx.at[k]], ssems[u], add=True)

            @pl.when(k >= 2)
            def _():
                pltpu.make_async_copy(bufs[v], acc.at[didx.at[k - 2]],
                                      ssems[v]).wait()

            @pl.when(k + 2 < n)
            def _():
                pltpu.async_copy(table.at[sidx.at[k + 2]], bufs[v], gsems[v])
        return 0

    lax.fori_loop(0, n // 4, quad, 0)
    pltpu.make_async_copy(bufs[2], acc.at[didx.at[n - 2]], ssems[2]).wait()
    pltpu.make_async_copy(bufs[3], acc.at[didx.at[n - 1]], ssems[3]).wait()


def _sc_body(h_hbm, src_hbm, dst_hbm, zero_hbm, agg_hbm, agg2_hbm,
             sidx, didx, b0, b1, b2, b3, acc,
             g0, g1, g2, g3, s0, s1, s2, s3):
    c = lax.axis_index("c")
    s = lax.axis_index("s")
    rows = pl.ds(s * ZROWS, ZROWS)
    bufs = (b0, b1, b2, b3)
    gsems = (g0, g1, g2, g3)
    ssems = (s0, s1, s2, s3)

    def full_pass(part, segs, out_view):
        pltpu.sync_copy(zero_hbm, acc.at[rows])
        plsc.subcore_barrier()
        for seg in segs:
            pltpu.sync_copy(src_hbm.at[s, pl.ds(seg * SEG, SEG)], sidx)
            pltpu.sync_copy(dst_hbm.at[s, pl.ds(seg * SEG, SEG)], didx)
            _gather_scatter_segment(h_hbm.at[part], sidx, didx, acc,
                                    bufs, gsems, ssems)
        plsc.subcore_barrier()
        pltpu.sync_copy(acc.at[rows], out_view.at[rows])

    @pl.when(c == 0)
    def _():
        full_pass(0, (0, 1, 2, 3), agg_hbm.at[0])
        full_pass(2, (0, 1), agg2_hbm.at[0])

    @pl.when(c == 1)
    def _():
        full_pass(1, (0, 1, 2, 3), agg_hbm.at[1])
        full_pass(2, (2, 3), agg2_hbm.at[1])


@functools.cache
def _sc_msg_kernel():
    return pl.kernel(
        _sc_body,
        out_type=(
            jax.ShapeDtypeStruct((2, NP_, PD), jnp.float32),
            jax.ShapeDtypeStruct((2, NP_, PD), jnp.float32),
        ),
        mesh=plsc.VectorSubcoreMesh(
            core_axis_name="c", subcore_axis_name="s",
            num_cores=NC, num_subcores=NS),
        scratch_types=(
            [pltpu.VMEM((SEG, CH), jnp.int32)] * 2
            + [pltpu.VMEM((CH, PD), jnp.float32)] * 4
            + [pltpu.VMEM_SHARED((NP_, PD), jnp.float32)]
            + [pltpu.SemaphoreType.DMA] * 8
        ),
    )


def _sc_msg(h, src_t, dst_t, zrows):
    return _sc_msg_kernel()(h, src_t, dst_t, zrows)


def _mlp_body(scale_ref, h_ref, agg_ref, agg2_ref, w1_ref, b1_ref, w2_ref,
              b2_ref, out_ref):
    scale = scale_ref[0, 0]
    hin = [scale * h_ref[0] + agg_ref[0],
           scale * h_ref[1] + agg_ref[1],
           scale * h_ref[2] + agg2_ref[0] + agg2_ref[1]]
    t = b1_ref[...]
    for p in range(NPART):
        t = t + jnp.dot(hin[p], w1_ref[p], preferred_element_type=jnp.float32)
    t = jnp.maximum(t, 0.0)
    for p in range(NPART):
        out_ref[p] = jnp.maximum(
            jnp.dot(t, w2_ref[p], preferred_element_type=jnp.float32)
            + b2_ref[p][None, :], 0.0)


def _mlp_call(scale, h, agg, agg2, w1, b1, w2, b2):
    return pl.pallas_call(
        _mlp_body,
        grid=(NP_ // B_TC,),
        in_specs=[
            pl.BlockSpec(memory_space=pltpu.SMEM),
            pl.BlockSpec((NPART, B_TC, PD), lambda i: (0, i, 0)),
            pl.BlockSpec((2, B_TC, PD), lambda i: (0, i, 0)),
            pl.BlockSpec((2, B_TC, PD), lambda i: (0, i, 0)),
            pl.BlockSpec((NPART, PD, H), lambda i: (0, 0, 0)),
            pl.BlockSpec((1, H), lambda i: (0, 0)),
            pl.BlockSpec((NPART, H, PD), lambda i: (0, 0, 0)),
            pl.BlockSpec((NPART, PD), lambda i: (0, 0)),
        ],
        out_specs=pl.BlockSpec((NPART, B_TC, PD), lambda i: (0, i, 0)),
        out_shape=jax.ShapeDtypeStruct((NPART, NP_, PD), jnp.float32),
    )(scale, h, agg, agg2, w1, b1, w2, b2)


def _head_body(h_ref, p_ref, wh_ref, bh_ref, out_ref, accs):
    i = pl.program_id(0)

    @pl.when(i == 0)
    def _():
        accs[...] = jnp.zeros_like(accs)

    for p in range(NPART):
        accs[p] += jnp.dot(p_ref[...], h_ref[p],
                           preferred_element_type=jnp.float32)

    @pl.when(i == NP_ // C_HD - 1)
    def _():
        cnt = accs[NPART - 1][:, PD - 1:PD]
        inv = 1.0 / jnp.maximum(cnt, 1.0)
        out = bh_ref[...]
        for p in range(NPART):
            out = out + jnp.dot(accs[p] * inv, wh_ref[p],
                                preferred_element_type=jnp.float32)
        out_ref[...] = out


def _head_call(h, p, wh, bh):
    return pl.pallas_call(
        _head_body,
        grid=(NP_ // C_HD,),
        in_specs=[
            pl.BlockSpec((NPART, C_HD, PD), lambda i: (0, i, 0)),
            pl.BlockSpec((G, C_HD), lambda i: (0, i)),
            pl.BlockSpec((NPART, PD, OUT), lambda i: (0, 0, 0)),
            pl.BlockSpec((1, OUT), lambda i: (0, 0)),
        ],
        out_specs=pl.BlockSpec((G, OUT), lambda i: (0, 0)),
        out_shape=jax.ShapeDtypeStruct((G, OUT), jnp.float32),
        scratch_shapes=[
            pltpu.VMEM((NPART, G, PD), jnp.float32),
        ],
    )(h, p, wh, bh)


def _part_pad(a, ncols_axis=-1):
    pads = [(0, 0)] * a.ndim
    pads[ncols_axis] = (0, NPART * PD - D)
    ap = jnp.pad(a, pads)
    return ap


def kernel(x, edge_index, graph_ids, W1, b1, W2, b2, eps, W_head, b_head):
    f32 = jnp.float32
    src = edge_index[0].astype(jnp.int32)
    dst = edge_index[1].astype(jnp.int32)

    src_t = jnp.full((NS, EPT), NP_ - 1, jnp.int32).at[:, :E // NS].set(
        src.reshape(NS, E // NS)).reshape(NS, NCHUNK, CH)
    dst_t = jnp.full((NS, EPT), N, jnp.int32).at[:, :E // NS].set(
        dst.reshape(NS, E // NS)).reshape(NS, NCHUNK, CH)

    xp = _part_pad(x)
    h = jnp.zeros((NPART, NP_, PD), f32)
    for p in range(NPART):
        h = h.at[p, :N].set(xp[:, p * PD:(p + 1) * PD])

    W1p = _part_pad(W1, 1).reshape(L, NPART, PD, H)
    W2p = _part_pad(W2).reshape(L, H, NPART, PD).transpose(0, 2, 1, 3)
    b2p = _part_pad(b2).reshape(L, NPART, PD)
    b1r = b1.reshape(L, 1, H)
    scales = (1.0 + eps).astype(f32).reshape(L, 1, 1)
    zrows = jnp.zeros((ZROWS, PD), f32)

    for l in range(L):
        agg, agg2 = _sc_msg(h, src_t, dst_t, zrows)
        h = _mlp_call(scales[l], h, agg, agg2,
                      W1p[l], b1r[l], W2p[l], b2p[l])

    onehot = (graph_ids[None, :] == jnp.arange(G, dtype=graph_ids.dtype)[:, None])
    pmat = jnp.zeros((G, NP_), f32).at[:, :N].set(onehot.astype(f32))
    hh = h.at[NPART - 1, :, PD - 1].set(1.0)
    whp = _part_pad(W_head, 0).reshape(NPART, PD, OUT)
    return _head_call(hh, pmat, whp, b_head.reshape(1, OUT))


# speedup vs baseline: 2.4766x; 1.0011x over previous
import functools

import jax
import jax.numpy as jnp
from jax import lax
from jax.experimental import pallas as pl
from jax.experimental.pallas import tpu as pltpu
from jax.experimental.pallas import tpu_sc as plsc

N = 10000
E = 160000
D = 300
H = 600
L = 5
G = 64
OUT = 2048

PD = 128
NPART = 3
NP_ = 10240
NC = 2
NS = 16
CH = 64
EPT = 10240
NCHUNK = EPT // CH
SEG = 40
ZROWS = NP_ // NS
B_TC = 512
C_HD = 1280


def _gather_scatter_segment(table, sidx, didx, acc, bufs, gsems, ssems):
    n = SEG
    pltpu.async_copy(table.at[sidx.at[0]], bufs[0], gsems[0])
    pltpu.async_copy(table.at[sidx.at[1]], bufs[1], gsems[1])

    def quad(i, _):
        for u in range(4):
            k = 4 * i + u
            v = (u + 2) % 4
            pltpu.make_async_copy(table.at[sidx.at[k]], bufs[u], gsems[u]).wait()
            pltpu.async_copy(bufs[u], acc.at[didx.at[k]], ssems[u], add=True)

            @pl.when(k >= 2)
            def _():
                pltpu.make_async_copy(bufs[v], acc.at[didx.at[k - 2]],
                                      ssems[v]).wait()

            @pl.when(k + 2 < n)
            def _():
                pltpu.async_copy(table.at[sidx.at[k + 2]], bufs[v], gsems[v])
        return 0

    lax.fori_loop(0, n // 4, quad, 0)
    pltpu.make_async_copy(bufs[2], acc.at[didx.at[n - 2]], ssems[2]).wait()
    pltpu.make_async_copy(bufs[3], acc.at[didx.at[n - 1]], ssems[3]).wait()


def _sc_body(h_hbm, src_hbm, dst_hbm, zero_hbm, agg_hbm, agg2_hbm,
             sidx, didx, b0, b1, b2, b3, acc,
             g0, g1, g2, g3, s0, s1, s2, s3):
    c = lax.axis_index("c")
    s = lax.axis_index("s")
    rows = pl.ds(s * ZROWS, ZROWS)
    bufs = (b0, b1, b2, b3)
    gsems = (g0, g1, g2, g3)
    ssems = (s0, s1, s2, s3)

    def full_pass(table, seg0, nseg, out_view):
        pltpu.sync_copy(zero_hbm, acc.at[rows])
        plsc.subcore_barrier()
        for j in range(nseg):
            seg = seg0 + j
            pltpu.sync_copy(src_hbm.at[s, pl.ds(seg * SEG, SEG)], sidx)
            pltpu.sync_copy(dst_hbm.at[s, pl.ds(seg * SEG, SEG)], didx)
            _gather_scatter_segment(table, sidx, didx, acc,
                                    bufs, gsems, ssems)
        plsc.subcore_barrier()
        pltpu.sync_copy(acc.at[rows], out_view.at[rows])

    full_pass(h_hbm.at[c], 0, 4, agg_hbm.at[c])
    full_pass(h_hbm.at[2], 2 * c, 2, agg2_hbm.at[c])


@functools.cache
def _sc_msg_kernel():
    return pl.kernel(
        _sc_body,
        out_type=(
            jax.ShapeDtypeStruct((2, NP_, PD), jnp.float32),
            jax.ShapeDtypeStruct((2, NP_, PD), jnp.float32),
        ),
        mesh=plsc.VectorSubcoreMesh(
            core_axis_name="c", subcore_axis_name="s",
            num_cores=NC, num_subcores=NS),
        scratch_types=(
            [pltpu.VMEM((SEG, CH), jnp.int32)] * 2
            + [pltpu.VMEM((CH, PD), jnp.float32)] * 4
            + [pltpu.VMEM_SHARED((NP_, PD), jnp.float32)]
            + [pltpu.SemaphoreType.DMA] * 8
        ),
    )


def _sc_msg(h, src_t, dst_t, zrows):
    return _sc_msg_kernel()(h, src_t, dst_t, zrows)


def _mlp_body(scale_ref, h_ref, agg_ref, agg2_ref, w1_ref, b1_ref, w2_ref,
              b2_ref, out_ref):
    scale = scale_ref[0, 0]
    hin = [scale * h_ref[0] + agg_ref[0],
           scale * h_ref[1] + agg_ref[1],
           scale * h_ref[2] + agg2_ref[0] + agg2_ref[1]]
    t = b1_ref[...]
    for p in range(NPART):
        t = t + jnp.dot(hin[p], w1_ref[p], preferred_element_type=jnp.float32)
    t = jnp.maximum(t, 0.0)
    for p in range(NPART):
        out_ref[p] = jnp.maximum(
            jnp.dot(t, w2_ref[p], preferred_element_type=jnp.float32)
            + b2_ref[p][None, :], 0.0)


def _mlp_call(scale, h, agg, agg2, w1, b1, w2, b2):
    return pl.pallas_call(
        _mlp_body,
        grid=(NP_ // B_TC,),
        in_specs=[
            pl.BlockSpec(memory_space=pltpu.SMEM),
            pl.BlockSpec((NPART, B_TC, PD), lambda i: (0, i, 0)),
            pl.BlockSpec((2, B_TC, PD), lambda i: (0, i, 0)),
            pl.BlockSpec((2, B_TC, PD), lambda i: (0, i, 0)),
            pl.BlockSpec((NPART, PD, H), lambda i: (0, 0, 0)),
            pl.BlockSpec((1, H), lambda i: (0, 0)),
            pl.BlockSpec((NPART, H, PD), lambda i: (0, 0, 0)),
            pl.BlockSpec((NPART, PD), lambda i: (0, 0)),
        ],
        out_specs=pl.BlockSpec((NPART, B_TC, PD), lambda i: (0, i, 0)),
        out_shape=jax.ShapeDtypeStruct((NPART, NP_, PD), jnp.float32),
    )(scale, h, agg, agg2, w1, b1, w2, b2)


def _head_body(h_ref, p_ref, wh_ref, bh_ref, out_ref, accs):
    i = pl.program_id(0)

    @pl.when(i == 0)
    def _():
        accs[...] = jnp.zeros_like(accs)

    for p in range(NPART):
        accs[p] += jnp.dot(p_ref[...], h_ref[p],
                           preferred_element_type=jnp.float32)

    @pl.when(i == NP_ // C_HD - 1)
    def _():
        cnt = accs[NPART - 1][:, PD - 1:PD]
        inv = 1.0 / jnp.maximum(cnt, 1.0)
        out = bh_ref[...]
        for p in range(NPART):
            out = out + jnp.dot(accs[p] * inv, wh_ref[p],
                                preferred_element_type=jnp.float32)
        out_ref[...] = out


def _head_call(h, p, wh, bh):
    return pl.pallas_call(
        _head_body,
        grid=(NP_ // C_HD,),
        in_specs=[
            pl.BlockSpec((NPART, C_HD, PD), lambda i: (0, i, 0)),
            pl.BlockSpec((G, C_HD), lambda i: (0, i)),
            pl.BlockSpec((NPART, PD, OUT), lambda i: (0, 0, 0)),
            pl.BlockSpec((1, OUT), lambda i: (0, 0)),
        ],
        out_specs=pl.BlockSpec((G, OUT), lambda i: (0, 0)),
        out_shape=jax.ShapeDtypeStruct((G, OUT), jnp.float32),
        scratch_shapes=[
            pltpu.VMEM((NPART, G, PD), jnp.float32),
        ],
    )(h, p, wh, bh)


def _part_pad(a, ncols_axis=-1):
    pads = [(0, 0)] * a.ndim
    pads[ncols_axis] = (0, NPART * PD - D)
    ap = jnp.pad(a, pads)
    return ap


def kernel(x, edge_index, graph_ids, W1, b1, W2, b2, eps, W_head, b_head):
    f32 = jnp.float32
    src = edge_index[0].astype(jnp.int32)
    dst = edge_index[1].astype(jnp.int32)

    src_t = jnp.full((NS, EPT), NP_ - 1, jnp.int32).at[:, :E // NS].set(
        src.reshape(NS, E // NS)).reshape(NS, NCHUNK, CH)
    dst_t = jnp.full((NS, EPT), N, jnp.int32).at[:, :E // NS].set(
        dst.reshape(NS, E // NS)).reshape(NS, NCHUNK, CH)

    xp = _part_pad(x)
    h = jnp.zeros((NPART, NP_, PD), f32)
    for p in range(NPART):
        h = h.at[p, :N].set(xp[:, p * PD:(p + 1) * PD])

    W1p = _part_pad(W1, 1).reshape(L, NPART, PD, H)
    W2p = _part_pad(W2).reshape(L, H, NPART, PD).transpose(0, 2, 1, 3)
    b2p = _part_pad(b2).reshape(L, NPART, PD)
    b1r = b1.reshape(L, 1, H)
    scales = (1.0 + eps).astype(f32).reshape(L, 1, 1)
    zrows = jnp.zeros((ZROWS, PD), f32)

    for l in range(L):
        agg, agg2 = _sc_msg(h, src_t, dst_t, zrows)
        h = _mlp_call(scales[l], h, agg, agg2,
                      W1p[l], b1r[l], W2p[l], b2p[l])

    onehot = (graph_ids[None, :] == jnp.arange(G, dtype=graph_ids.dtype)[:, None])
    pmat = jnp.zeros((G, NP_), f32).at[:, :N].set(onehot.astype(f32))
    hh = h.at[NPART - 1, :, PD - 1].set(1.0)
    whp = _part_pad(W_head, 0).reshape(NPART, PD, OUT)
    return _head_call(hh, pmat, whp, b_head.reshape(1, OUT))


# speedup vs baseline: 2.5210x; 1.0179x over previous
import functools

import jax
import jax.numpy as jnp
from jax import lax
from jax.experimental import pallas as pl
from jax.experimental.pallas import tpu as pltpu
from jax.experimental.pallas import tpu_sc as plsc

N = 10000
E = 160000
D = 300
H = 600
L = 5
G = 64
OUT = 2048

PD = 128
NPART = 3
NP_ = 10240
NC = 2
NS = 16
CH = 64
EPT = 10240
NCHUNK = EPT // CH
SEG = 40
ZROWS = NP_ // NS
B_TC = 512
C_HD = 1280


def _gather_scatter_segment(table, sidx, didx, acc, bufs, gsems, ssems):
    n = SEG
    pltpu.async_copy(table.at[sidx.at[0]], bufs[0], gsems[0])
    pltpu.async_copy(table.at[sidx.at[1]], bufs[1], gsems[1])

    def quad(i, _):
        for u in range(4):
            k = 4 * i + u
            v = (u + 2) % 4
            pltpu.make_async_copy(table.at[sidx.at[k]], bufs[u], gsems[u]).wait()
            pltpu.async_copy(bufs[u], acc.at[didx.at[k]], ssems[u], add=True)

            @pl.when(k >= 2)
            def _():
                pltpu.make_async_copy(bufs[v], acc.at[didx.at[k - 2]],
                                      ssems[v]).wait()

            @pl.when(k + 2 < n)
            def _():
                pltpu.async_copy(table.at[sidx.at[k + 2]], bufs[v], gsems[v])
        return 0

    lax.fori_loop(0, n // 4, quad, 0)
    pltpu.make_async_copy(bufs[2], acc.at[didx.at[n - 2]], ssems[2]).wait()
    pltpu.make_async_copy(bufs[3], acc.at[didx.at[n - 1]], ssems[3]).wait()


def _sc_body(h_hbm, src_hbm, dst_hbm, zero_hbm, agg_hbm, agg2_hbm,
             sidx, didx, b0, b1, b2, b3, acc,
             g0, g1, g2, g3, s0, s1, s2, s3):
    c = lax.axis_index("c")
    s = lax.axis_index("s")
    rows = pl.ds(s * ZROWS, ZROWS)
    bufs = (b0, b1, b2, b3)
    gsems = (g0, g1, g2, g3)
    ssems = (s0, s1, s2, s3)

    def segment(table, seg):
        pltpu.sync_copy(src_hbm.at[s, pl.ds(seg * SEG, SEG)], sidx)
        pltpu.sync_copy(dst_hbm.at[s, pl.ds(seg * SEG, SEG)], didx)
        _gather_scatter_segment(table, sidx, didx, acc,
                                bufs, gsems, ssems)

    def prologue():
        pltpu.sync_copy(zero_hbm, acc.at[rows])
        plsc.subcore_barrier()

    def epilogue(out_view):
        plsc.subcore_barrier()
        pltpu.sync_copy(acc.at[rows], out_view.at[rows])

    prologue()
    for j in range(4):
        segment(h_hbm.at[c], j)
    epilogue(agg_hbm.at[c])

    prologue()
    for j in range(3):
        seg = 3 * c + j

        @pl.when((c == 0) | (j == 0))
        def _():
            segment(h_hbm.at[2], seg)
    epilogue(agg2_hbm.at[c])


@functools.cache
def _sc_msg_kernel():
    return pl.kernel(
        _sc_body,
        out_type=(
            jax.ShapeDtypeStruct((2, NP_, PD), jnp.float32),
            jax.ShapeDtypeStruct((2, NP_, PD), jnp.float32),
        ),
        mesh=plsc.VectorSubcoreMesh(
            core_axis_name="c", subcore_axis_name="s",
            num_cores=NC, num_subcores=NS),
        scratch_types=(
            [pltpu.VMEM((SEG, CH), jnp.int32)] * 2
            + [pltpu.VMEM((CH, PD), jnp.float32)] * 4
            + [pltpu.VMEM_SHARED((NP_, PD), jnp.float32)]
            + [pltpu.SemaphoreType.DMA] * 8
        ),
    )


def _sc_msg(h, src_t, dst_t, zrows):
    return _sc_msg_kernel()(h, src_t, dst_t, zrows)


def _mlp_body(scale_ref, h_ref, agg_ref, agg2_ref, w1_ref, b1_ref, w2_ref,
              b2_ref, out_ref):
    scale = scale_ref[0, 0]
    hin = [scale * h_ref[0] + agg_ref[0],
           scale * h_ref[1] + agg_ref[1],
           scale * h_ref[2] + agg2_ref[0] + agg2_ref[1]]
    t = b1_ref[...]
    for p in range(NPART):
        t = t + jnp.dot(hin[p], w1_ref[p], preferred_element_type=jnp.float32)
    t = jnp.maximum(t, 0.0)
    for p in range(NPART):
        out_ref[p] = jnp.maximum(
            jnp.dot(t, w2_ref[p], preferred_element_type=jnp.float32)
            + b2_ref[p][None, :], 0.0)


def _mlp_call(scale, h, agg, agg2, w1, b1, w2, b2):
    return pl.pallas_call(
        _mlp_body,
        grid=(NP_ // B_TC,),
        in_specs=[
            pl.BlockSpec(memory_space=pltpu.SMEM),
            pl.BlockSpec((NPART, B_TC, PD), lambda i: (0, i, 0)),
            pl.BlockSpec((2, B_TC, PD), lambda i: (0, i, 0)),
            pl.BlockSpec((2, B_TC, PD), lambda i: (0, i, 0)),
            pl.BlockSpec((NPART, PD, H), lambda i: (0, 0, 0)),
            pl.BlockSpec((1, H), lambda i: (0, 0)),
            pl.BlockSpec((NPART, H, PD), lambda i: (0, 0, 0)),
            pl.BlockSpec((NPART, PD), lambda i: (0, 0)),
        ],
        out_specs=pl.BlockSpec((NPART, B_TC, PD), lambda i: (0, i, 0)),
        out_shape=jax.ShapeDtypeStruct((NPART, NP_, PD), jnp.float32),
    )(scale, h, agg, agg2, w1, b1, w2, b2)


def _head_body(h_ref, p_ref, wh_ref, bh_ref, out_ref, accs):
    i = pl.program_id(0)

    @pl.when(i == 0)
    def _():
        accs[...] = jnp.zeros_like(accs)

    for p in range(NPART):
        accs[p] += jnp.dot(p_ref[...], h_ref[p],
                           preferred_element_type=jnp.float32)

    @pl.when(i == NP_ // C_HD - 1)
    def _():
        cnt = accs[NPART - 1][:, PD - 1:PD]
        inv = 1.0 / jnp.maximum(cnt, 1.0)
        out = bh_ref[...]
        for p in range(NPART):
            out = out + jnp.dot(accs[p] * inv, wh_ref[p],
                                preferred_element_type=jnp.float32)
        out_ref[...] = out


def _head_call(h, p, wh, bh):
    return pl.pallas_call(
        _head_body,
        grid=(NP_ // C_HD,),
        in_specs=[
            pl.BlockSpec((NPART, C_HD, PD), lambda i: (0, i, 0)),
            pl.BlockSpec((G, C_HD), lambda i: (0, i)),
            pl.BlockSpec((NPART, PD, OUT), lambda i: (0, 0, 0)),
            pl.BlockSpec((1, OUT), lambda i: (0, 0)),
        ],
        out_specs=pl.BlockSpec((G, OUT), lambda i: (0, 0)),
        out_shape=jax.ShapeDtypeStruct((G, OUT), jnp.float32),
        scratch_shapes=[
            pltpu.VMEM((NPART, G, PD), jnp.float32),
        ],
    )(h, p, wh, bh)


def _part_pad(a, ncols_axis=-1):
    pads = [(0, 0)] * a.ndim
    pads[ncols_axis] = (0, NPART * PD - D)
    ap = jnp.pad(a, pads)
    return ap


def kernel(x, edge_index, graph_ids, W1, b1, W2, b2, eps, W_head, b_head):
    f32 = jnp.float32
    src = edge_index[0].astype(jnp.int32)
    dst = edge_index[1].astype(jnp.int32)

    src_t = jnp.full((NS, EPT), NP_ - 1, jnp.int32).at[:, :E // NS].set(
        src.reshape(NS, E // NS)).reshape(NS, NCHUNK, CH)
    dst_t = jnp.full((NS, EPT), N, jnp.int32).at[:, :E // NS].set(
        dst.reshape(NS, E // NS)).reshape(NS, NCHUNK, CH)

    xp = _part_pad(x)
    h = jnp.zeros((NPART, NP_, PD), f32)
    for p in range(NPART):
        h = h.at[p, :N].set(xp[:, p * PD:(p + 1) * PD])

    W1p = _part_pad(W1, 1).reshape(L, NPART, PD, H)
    W2p = _part_pad(W2).reshape(L, H, NPART, PD).transpose(0, 2, 1, 3)
    b2p = _part_pad(b2).reshape(L, NPART, PD)
    b1r = b1.reshape(L, 1, H)
    scales = (1.0 + eps).astype(f32).reshape(L, 1, 1)
    zrows = jnp.zeros((ZROWS, PD), f32)

    for l in range(L):
        agg, agg2 = _sc_msg(h, src_t, dst_t, zrows)
        h = _mlp_call(scales[l], h, agg, agg2,
                      W1p[l], b1r[l], W2p[l], b2p[l])

    onehot = (graph_ids[None, :] == jnp.arange(G, dtype=graph_ids.dtype)[:, None])
    pmat = jnp.zeros((G, NP_), f32).at[:, :N].set(onehot.astype(f32))
    hh = h.at[NPART - 1, :, PD - 1].set(1.0)
    whp = _part_pad(W_head, 0).reshape(NPART, PD, OUT)
    return _head_call(hh, pmat, whp, b_head.reshape(1, OUT))
